# baseline (speedup 1.0000x reference)
import jax
import jax.numpy as jnp
from jax import lax
from jax.experimental import pallas as pl
from jax.experimental.pallas import tpu as pltpu
from jax.experimental.pallas import tpu_sc as plsc

ROWS = 16384
COLS = 512
LANES = 16
NUM_GROUPS = COLS // LANES

NUM_CORES = 2
NUM_SUBCORES = 16
NUM_WORKERS = NUM_CORES * NUM_SUBCORES
ROWS_PER_WORKER = ROWS // NUM_WORKERS

CHUNK_ROWS = 32
NUM_CHUNKS = ROWS_PER_WORKER // CHUNK_ROWS


def _permute_body(x_hbm, perm_hbm, out_hbm, perm_v,
                  in_v0, in_v1, out_v0, out_v1,
                  sem_in0, sem_in1, sem_out0, sem_out1):
    wid = lax.axis_index("s") * NUM_CORES + lax.axis_index("c")
    base = wid * ROWS_PER_WORKER

    in_bufs = (in_v0, in_v1)
    out_bufs = (out_v0, out_v1)
    in_sems = (sem_in0, sem_in1)
    out_sems = (sem_out0, sem_out1)

    pltpu.sync_copy(perm_hbm, perm_v)
    idx_vecs = [perm_v[pl.ds(g * LANES, LANES)] for g in range(NUM_GROUPS)]

    def x_slice(c):
        return x_hbm.at[pl.ds(base + c * CHUNK_ROWS, CHUNK_ROWS)]

    def out_slice(c):
        return out_hbm.at[pl.ds(base + c * CHUNK_ROWS, CHUNK_ROWS)]

    in_dma = {}
    out_dma = {}
    in_dma[0] = pltpu.async_copy(x_slice(0), in_bufs[0], in_sems[0])
    for c in range(NUM_CHUNKS):
        b = c % 2
        if c + 1 < NUM_CHUNKS:
            in_dma[c + 1] = pltpu.async_copy(
                x_slice(c + 1), in_bufs[1 - b], in_sems[1 - b])
        in_dma[c].wait()
        if c >= 2:
            out_dma[c - 2].wait()
        in_v, out_v = in_bufs[b], out_bufs[b]

        @plsc.parallel_loop(0, CHUNK_ROWS)
        def row_body(r):
            rvec = jnp.full((LANES,), r, dtype=jnp.int32)
            for g in range(NUM_GROUPS):
                vals = plsc.load_gather(in_v, [rvec, idx_vecs[g]])
                out_v[r, pl.ds(g * LANES, LANES)] = vals

        out_dma[c] = pltpu.async_copy(out_v, out_slice(c), out_sems[b])

    out_dma[NUM_CHUNKS - 2].wait()
    out_dma[NUM_CHUNKS - 1].wait()


_permute = pl.kernel(
    _permute_body,
    out_type=jax.ShapeDtypeStruct((ROWS, COLS), jnp.float32),
    mesh=plsc.VectorSubcoreMesh(
        core_axis_name="c", subcore_axis_name="s",
        num_cores=NUM_CORES, num_subcores=NUM_SUBCORES,
    ),
    scratch_types=[
        pltpu.VMEM((COLS,), jnp.int32),
        pltpu.VMEM((CHUNK_ROWS, COLS), jnp.float32),
        pltpu.VMEM((CHUNK_ROWS, COLS), jnp.float32),
        pltpu.VMEM((CHUNK_ROWS, COLS), jnp.float32),
        pltpu.VMEM((CHUNK_ROWS, COLS), jnp.float32),
        pltpu.SemaphoreType.DMA,
        pltpu.SemaphoreType.DMA,
        pltpu.SemaphoreType.DMA,
        pltpu.SemaphoreType.DMA,
    ],
    compiler_params=pltpu.CompilerParams(
        use_tc_tiling_on_sc=True, needs_layout_passes=False,
    ),
)


def kernel(x, permutation):
    return _permute(x, permutation)


# speedup vs baseline: 1.0223x; 1.0223x over previous
import jax
import jax.numpy as jnp
from jax import lax
from jax.experimental import pallas as pl
from jax.experimental.pallas import tpu as pltpu
from jax.experimental.pallas import tpu_sc as plsc

ROWS = 16384
COLS = 512
LANES = 16
NUM_GROUPS = COLS // LANES

NUM_CORES = 2
NUM_SUBCORES = 16
NUM_WORKERS = NUM_CORES * NUM_SUBCORES
ROWS_PER_WORKER = ROWS // NUM_WORKERS

CHUNK_ROWS = 32
NUM_CHUNKS = ROWS_PER_WORKER // CHUNK_ROWS


def _permute_body(x_hbm, perm_hbm, out_hbm, perm_v,
                  in_v0, in_v1, in_v2, out_v0, out_v1,
                  sem_in0, sem_in1, sem_in2, sem_out0, sem_out1):
    wid = lax.axis_index("s") * NUM_CORES + lax.axis_index("c")
    base = wid * ROWS_PER_WORKER

    in_bufs = (in_v0, in_v1, in_v2)
    out_bufs = (out_v0, out_v1)
    in_sems = (sem_in0, sem_in1, sem_in2)
    out_sems = (sem_out0, sem_out1)

    pltpu.sync_copy(perm_hbm, perm_v)
    idx_vecs = [perm_v[pl.ds(g * LANES, LANES)] for g in range(NUM_GROUPS)]

    def x_slice(c):
        return x_hbm.at[pl.ds(base + c * CHUNK_ROWS, CHUNK_ROWS)]

    def out_slice(c):
        return out_hbm.at[pl.ds(base + c * CHUNK_ROWS, CHUNK_ROWS)]

    in_dma = {}
    out_dma = {}
    in_dma[0] = pltpu.async_copy(x_slice(0), in_bufs[0], in_sems[0])
    in_dma[1] = pltpu.async_copy(x_slice(1), in_bufs[1], in_sems[1])
    for c in range(NUM_CHUNKS):
        bi = c % 3
        b = c % 2
        if c + 2 < NUM_CHUNKS:
            in_dma[c + 2] = pltpu.async_copy(
                x_slice(c + 2), in_bufs[(c + 2) % 3], in_sems[(c + 2) % 3])
        in_dma[c].wait()
        if c >= 2:
            out_dma[c - 2].wait()
        in_v, out_v = in_bufs[bi], out_bufs[b]

        @plsc.parallel_loop(0, CHUNK_ROWS)
        def row_body(r):
            rvec = jnp.full((LANES,), r, dtype=jnp.int32)
            for g in range(NUM_GROUPS):
                vals = plsc.load_gather(in_v, [rvec, idx_vecs[g]])
                out_v[r, pl.ds(g * LANES, LANES)] = vals

        out_dma[c] = pltpu.async_copy(out_v, out_slice(c), out_sems[b])

    out_dma[NUM_CHUNKS - 2].wait()
    out_dma[NUM_CHUNKS - 1].wait()


_permute = pl.kernel(
    _permute_body,
    out_type=jax.ShapeDtypeStruct((ROWS, COLS), jnp.float32),
    mesh=plsc.VectorSubcoreMesh(
        core_axis_name="c", subcore_axis_name="s",
        num_cores=NUM_CORES, num_subcores=NUM_SUBCORES,
    ),
    scratch_types=[
        pltpu.VMEM((COLS,), jnp.int32),
        pltpu.VMEM((CHUNK_ROWS, COLS), jnp.float32),
        pltpu.VMEM((CHUNK_ROWS, COLS), jnp.float32),
        pltpu.VMEM((CHUNK_ROWS, COLS), jnp.float32),
        pltpu.VMEM((CHUNK_ROWS, COLS), jnp.float32),
        pltpu.VMEM((CHUNK_ROWS, COLS), jnp.float32),
        pltpu.SemaphoreType.DMA,
        pltpu.SemaphoreType.DMA,
        pltpu.SemaphoreType.DMA,
        pltpu.SemaphoreType.DMA,
        pltpu.SemaphoreType.DMA,
    ],
    compiler_params=pltpu.CompilerParams(
        use_tc_tiling_on_sc=True, needs_layout_passes=False,
    ),
)


def kernel(x, permutation):
    return _permute(x, permutation)


# speedup vs baseline: 1.0287x; 1.0062x over previous
import jax
import jax.numpy as jnp
from jax import lax
from jax.experimental import pallas as pl
from jax.experimental.pallas import tpu as pltpu
from jax.experimental.pallas import tpu_sc as plsc

ROWS = 16384
COLS = 512
LANES = 16
NUM_GROUPS = COLS // LANES

NUM_CORES = 2
NUM_SUBCORES = 16
NUM_WORKERS = NUM_CORES * NUM_SUBCORES
ROWS_PER_WORKER = ROWS // NUM_WORKERS

CHUNK_ROWS = 32
NUM_CHUNKS = ROWS_PER_WORKER // CHUNK_ROWS


def _permute_body(x_hbm, perm_hbm, out_hbm, perm_v,
                  in_v0, in_v1, in_v2, in_v3, out_v0, out_v1, out_v2,
                  sem_in0, sem_in1, sem_in2, sem_in3,
                  sem_out0, sem_out1, sem_out2):
    wid = lax.axis_index("s") * NUM_CORES + lax.axis_index("c")
    base = wid * ROWS_PER_WORKER

    in_bufs = (in_v0, in_v1, in_v2, in_v3)
    out_bufs = (out_v0, out_v1, out_v2)
    in_sems = (sem_in0, sem_in1, sem_in2, sem_in3)
    out_sems = (sem_out0, sem_out1, sem_out2)

    pltpu.sync_copy(perm_hbm, perm_v)
    idx_vecs = [perm_v[pl.ds(g * LANES, LANES)] for g in range(NUM_GROUPS)]

    def x_slice(c):
        return x_hbm.at[pl.ds(base + c * CHUNK_ROWS, CHUNK_ROWS)]

    def out_slice(c):
        return out_hbm.at[pl.ds(base + c * CHUNK_ROWS, CHUNK_ROWS)]

    in_dma = {}
    out_dma = {}
    in_dma[0] = pltpu.async_copy(x_slice(0), in_bufs[0], in_sems[0])
    in_dma[1] = pltpu.async_copy(x_slice(1), in_bufs[1], in_sems[1])
    in_dma[2] = pltpu.async_copy(x_slice(2), in_bufs[2], in_sems[2])
    for c in range(NUM_CHUNKS):
        bi = c % 4
        b = c % 3
        if c + 3 < NUM_CHUNKS:
            in_dma[c + 3] = pltpu.async_copy(
                x_slice(c + 3), in_bufs[(c + 3) % 4], in_sems[(c + 3) % 4])
        in_dma[c].wait()
        if c >= 3:
            out_dma[c - 3].wait()
        in_v, out_v = in_bufs[bi], out_bufs[b]

        @plsc.parallel_loop(0, CHUNK_ROWS)
        def row_body(r):
            rvec = jnp.full((LANES,), r, dtype=jnp.int32)
            for g in range(NUM_GROUPS):
                vals = plsc.load_gather(in_v, [rvec, idx_vecs[g]])
                out_v[r, pl.ds(g * LANES, LANES)] = vals

        out_dma[c] = pltpu.async_copy(out_v, out_slice(c), out_sems[b])

    out_dma[NUM_CHUNKS - 3].wait()
    out_dma[NUM_CHUNKS - 2].wait()
    out_dma[NUM_CHUNKS - 1].wait()


_permute = pl.kernel(
    _permute_body,
    out_type=jax.ShapeDtypeStruct((ROWS, COLS), jnp.float32),
    mesh=plsc.VectorSubcoreMesh(
        core_axis_name="c", subcore_axis_name="s",
        num_cores=NUM_CORES, num_subcores=NUM_SUBCORES,
    ),
    scratch_types=[
        pltpu.VMEM((COLS,), jnp.int32),
        pltpu.VMEM((CHUNK_ROWS, COLS), jnp.float32),
        pltpu.VMEM((CHUNK_ROWS, COLS), jnp.float32),
        pltpu.VMEM((CHUNK_ROWS, COLS), jnp.float32),
        pltpu.VMEM((CHUNK_ROWS, COLS), jnp.float32),
        pltpu.VMEM((CHUNK_ROWS, COLS), jnp.float32),
        pltpu.VMEM((CHUNK_ROWS, COLS), jnp.float32),
        pltpu.VMEM((CHUNK_ROWS, COLS), jnp.float32),
        pltpu.SemaphoreType.DMA,
        pltpu.SemaphoreType.DMA,
        pltpu.SemaphoreType.DMA,
        pltpu.SemaphoreType.DMA,
        pltpu.SemaphoreType.DMA,
        pltpu.SemaphoreType.DMA,
        pltpu.SemaphoreType.DMA,
    ],
    compiler_params=pltpu.CompilerParams(
        use_tc_tiling_on_sc=True, needs_layout_passes=False,
    ),
)


def kernel(x, permutation):
    return _permute(x, permutation)
